# baseline (speedup 1.0000x reference)
import functools

import jax
import jax.numpy as jnp
from jax import lax
from jax.experimental import pallas as pl
from jax.experimental.pallas import tpu as pltpu
from jax.experimental.pallas import tpu_sc as plsc

_N_FIELDS = 26
_VOCAB = 100000
_EDIM = 32
_BATCH = 16384
_NPLANES = _N_FIELDS * _EDIM
_L = 16
_QBOUNDS = (0, 25088, 50176, 74880, _VOCAB)
_QMAX = 25120
_NQ = 4
_PACK = _BATCH
_UN = 4

_mesh = plsc.VectorSubcoreMesh(core_axis_name="c", subcore_axis_name="s")


@functools.partial(
    pl.kernel,
    mesh=_mesh,
    out_type=jax.ShapeDtypeStruct((_NPLANES, _BATCH), jnp.float32),
    compiler_params=pltpu.CompilerParams(
        use_tc_tiling_on_sc=True, needs_layout_passes=False
    ),
    scratch_types=[
        pltpu.VMEM((_QMAX,), jnp.float32),
        pltpu.VMEM((_QMAX,), jnp.float32),
        pltpu.VMEM((_BATCH,), jnp.int32),
        pltpu.VMEM((_BATCH + 2 * _UN * _L,), jnp.int32),
        pltpu.VMEM((_BATCH,), jnp.float32),
        pltpu.SemaphoreType.DMA,
        pltpu.SemaphoreType.DMA,
    ],
)
def _mk_gather(xt_hbm, tt_hbm, out_hbm, qa_v, qb_v, xv, pk_v, row_v, sema, semb):
    f = lax.axis_index("s") * 2 + lax.axis_index("c")
    lanes = lax.iota(jnp.int32, _L)

    @pl.when(f < _N_FIELDS)
    def _():
        pltpu.sync_copy(xt_hbm.at[f], xv)

        def count_grp(g, accs):
            a1, a2, a3 = accs
            v = xv[pl.ds(g * _L, _L)]
            a1 = a1 + jnp.where(v < _QBOUNDS[1], 1, 0)
            a2 = a2 + jnp.where(v < _QBOUNDS[2], 1, 0)
            a3 = a3 + jnp.where(v < _QBOUNDS[3], 1, 0)
            return a1, a2, a3

        zero = jnp.zeros((_L,), jnp.int32)
        a1, a2, a3 = lax.fori_loop(
            0, _BATCH // _L, count_grp, (zero, zero, zero)
        )
        b1 = jnp.sum(a1)
        b2 = jnp.sum(a2)
        b3 = jnp.sum(a3)

        def place_grp(g, offs):
            o0, o1, o2, o3 = offs
            v = xv[pl.ds(g * _L, _L)]
            p = v * _PACK + (lanes + g * _L)
            m0 = v < _QBOUNDS[1]
            m1 = jnp.logical_and(v >= _QBOUNDS[1], v < _QBOUNDS[2])
            m2 = jnp.logical_and(v >= _QBOUNDS[2], v < _QBOUNDS[3])
            m3 = v >= _QBOUNDS[3]
            plsc.store_compressed(pk_v.at[pl.ds(o0, _L)], p, mask=m0)
            plsc.store_compressed(pk_v.at[pl.ds(o1, _L)], p, mask=m1)
            plsc.store_compressed(pk_v.at[pl.ds(o2, _L)], p, mask=m2)
            plsc.store_compressed(pk_v.at[pl.ds(o3, _L)], p, mask=m3)
            o0 = o0 + jnp.sum(jnp.where(m0, 1, 0))
            o1 = o1 + jnp.sum(jnp.where(m1, 1, 0))
            o2 = o2 + jnp.sum(jnp.where(m2, 1, 0))
            o3 = o3 + jnp.sum(jnp.where(m3, 1, 0))
            return o0, o1, o2, o3

        lax.fori_loop(0, _BATCH // _L, place_grp, (0, b1, b2, b3))
        bounds = (0, b1, b2, b3, _BATCH)

        def gather_bucket(qbuf, qlo, s, e):
            def grp(k, carry2):
                base_o = s + k * (_UN * _L)
                for u in range(_UN):
                    o = base_o + u * _L
                    pvec = pk_v[pl.ds(o, _L)]
                    m = (lanes + o) < e
                    idx = lax.shift_right_logical(pvec, 14) - qlo
                    pos = lax.bitwise_and(pvec, _PACK - 1)
                    val = plsc.load_gather(qbuf, [idx], mask=m)
                    plsc.store_scatter(row_v, [pos], val, mask=m)
                return carry2

            n_grp = (e - s + _UN * _L - 1) // (_UN * _L)
            lax.fori_loop(0, n_grp, grp, 0)

        bufs = (qa_v, qb_v)
        sems = (sema, semb)
        for e in range(_EDIM):
            copies = [
                pltpu.async_copy(
                    tt_hbm.at[f, e, pl.ds(_QBOUNDS[0], _QBOUNDS[1])],
                    bufs[0].at[pl.ds(0, _QBOUNDS[1])],
                    sems[0],
                )
            ]
            for q in range(_NQ):
                if q + 1 < _NQ:
                    qsz = _QBOUNDS[q + 2] - _QBOUNDS[q + 1]
                    copies.append(
                        pltpu.async_copy(
                            tt_hbm.at[f, e, pl.ds(_QBOUNDS[q + 1], qsz)],
                            bufs[(q + 1) % 2].at[pl.ds(0, qsz)],
                            sems[(q + 1) % 2],
                        )
                    )
                copies[q].wait()
                gather_bucket(
                    bufs[q % 2], _QBOUNDS[q], bounds[q], bounds[q + 1]
                )
            pltpu.sync_copy(row_v, out_hbm.at[f * _EDIM + e])


def kernel(x, tables):
    xt = x.T
    tt = tables.transpose(0, 2, 1)
    out = _mk_gather(xt, tt)
    return out.T


# speedup vs baseline: 1.2795x; 1.2795x over previous
import functools

import jax
import jax.numpy as jnp
from jax import lax
from jax.experimental import pallas as pl
from jax.experimental.pallas import tpu as pltpu
from jax.experimental.pallas import tpu_sc as plsc

_N_FIELDS = 26
_VOCAB = 100000
_EDIM = 32
_BATCH = 16384
_NW = 32
_NPLANES = _N_FIELDS * _EDIM
_PLANES_PER_W = _NPLANES // _NW
_LANES = 16
_UNROLL = 8
_BCHUNK = 4096

_mesh = plsc.VectorSubcoreMesh(core_axis_name="c", subcore_axis_name="s")


@functools.partial(
    pl.kernel,
    mesh=_mesh,
    out_type=jax.ShapeDtypeStruct((_NPLANES, _BATCH), jnp.float32),
    compiler_params=pltpu.CompilerParams(
        use_tc_tiling_on_sc=True, needs_layout_passes=False
    ),
    scratch_types=[
        pltpu.VMEM((_VOCAB,), jnp.float32),
        pltpu.VMEM((_BATCH,), jnp.int32),
        pltpu.VMEM((_BCHUNK,), jnp.float32),
    ],
)
def _mk_gather(xt_hbm, tt_hbm, out_hbm, plane_v, xv, row_v):
    wid = lax.axis_index("s") * 2 + lax.axis_index("c")

    def do_plane(j, carry):
        c = wid * _PLANES_PER_W + j
        f = c // _EDIM
        e = lax.rem(c, _EDIM)

        @pl.when(jnp.logical_or(j == 0, e == 0))
        def _():
            pltpu.sync_copy(xt_hbm.at[f], xv)

        pltpu.sync_copy(tt_hbm.at[f, e], plane_v)

        def do_bchunk(b, carry2):
            b0 = b * _BCHUNK

            def gather_group(i, carry3):
                base = i * (_LANES * _UNROLL)
                for k in range(_UNROLL):
                    o = base + k * _LANES
                    idx = xv[pl.ds(b0 + o, _LANES)]
                    row_v[pl.ds(o, _LANES)] = plsc.load_gather(plane_v, [idx])
                return carry3

            lax.fori_loop(0, _BCHUNK // (_LANES * _UNROLL), gather_group, 0)
            pltpu.sync_copy(row_v, out_hbm.at[c, pl.ds(b0, _BCHUNK)])
            return carry2

        lax.fori_loop(0, _BATCH // _BCHUNK, do_bchunk, 0)
        return carry

    lax.fori_loop(0, _PLANES_PER_W, do_plane, 0)


def kernel(x, tables):
    xt = x.T
    tt = tables.transpose(0, 2, 1)
    out = _mk_gather(xt, tt)
    return out.T


# speedup vs baseline: 1.4192x; 1.1092x over previous
import functools

import jax
import jax.numpy as jnp
from jax import lax
from jax.experimental import pallas as pl
from jax.experimental.pallas import tpu as pltpu
from jax.experimental.pallas import tpu_sc as plsc

_N_FIELDS = 26
_VOCAB = 100000
_EDIM = 32
_BATCH = 16384
_NW = 32
_NPLANES = _N_FIELDS * _EDIM
_PLANES_PER_W = _NPLANES // _NW
_LANES = 16
_UNROLL = 16
_BCHUNK = 4096

_mesh = plsc.VectorSubcoreMesh(core_axis_name="c", subcore_axis_name="s")


@functools.partial(
    pl.kernel,
    mesh=_mesh,
    out_type=jax.ShapeDtypeStruct((_NPLANES, _BATCH), jnp.float32),
    compiler_params=pltpu.CompilerParams(
        use_tc_tiling_on_sc=True, needs_layout_passes=False
    ),
    scratch_types=[
        pltpu.VMEM((_VOCAB,), jnp.float32),
        pltpu.VMEM((_BCHUNK,), jnp.int32),
        pltpu.VMEM((_BCHUNK,), jnp.float32),
    ],
)
def _mk_gather(xt_hbm, tt_hbm, out_hbm, plane_v, xv, row_v):
    wid = lax.axis_index("s") * 2 + lax.axis_index("c")

    def do_plane(j, carry):
        c = wid * _PLANES_PER_W + j
        f = c // _EDIM
        e = lax.rem(c, _EDIM)
        pltpu.sync_copy(tt_hbm.at[f, e], plane_v)

        def do_bchunk(b, carry2):
            b0 = b * _BCHUNK
            pltpu.sync_copy(xt_hbm.at[f, pl.ds(b0, _BCHUNK)], xv)

            def gather_group(i, carry3):
                base = i * (_LANES * _UNROLL)
                for k in range(_UNROLL):
                    o = base + k * _LANES
                    idx = xv[pl.ds(o, _LANES)]
                    row_v[pl.ds(o, _LANES)] = plsc.load_gather(plane_v, [idx])
                return carry3

            lax.fori_loop(0, _BCHUNK // (_LANES * _UNROLL), gather_group, 0)
            pltpu.sync_copy(row_v, out_hbm.at[c, pl.ds(b0, _BCHUNK)])
            return carry2

        lax.fori_loop(0, _BATCH // _BCHUNK, do_bchunk, 0)
        return carry

    lax.fori_loop(0, _PLANES_PER_W, do_plane, 0)


def kernel(x, tables):
    xt = x.T
    tt = tables.transpose(0, 2, 1)
    out = _mk_gather(xt, tt)
    return out.T


# speedup vs baseline: 1.9387x; 1.3661x over previous
import functools

import jax
import jax.numpy as jnp
from jax import lax
from jax.experimental import pallas as pl
from jax.experimental.pallas import tpu as pltpu
from jax.experimental.pallas import tpu_sc as plsc

_N_FIELDS = 26
_VOCAB = 100000
_EDIM = 32
_BATCH = 16384
_NW = 32
_NPLANES = _N_FIELDS * _EDIM
_PLANES_PER_W = _NPLANES // _NW
_LANES = 16
_UNROLL = 16
_BCHUNK = 4096
_NB = _BATCH // _BCHUNK

_mesh = plsc.VectorSubcoreMesh(core_axis_name="c", subcore_axis_name="s")


@functools.partial(
    pl.kernel,
    mesh=_mesh,
    out_type=jax.ShapeDtypeStruct((_NPLANES, _BATCH), jnp.float32),
    compiler_params=pltpu.CompilerParams(
        use_tc_tiling_on_sc=True, needs_layout_passes=False
    ),
    scratch_types=[
        pltpu.VMEM((_VOCAB,), jnp.float32),
        pltpu.VMEM((_BCHUNK,), jnp.int32),
        pltpu.VMEM((_BCHUNK,), jnp.int32),
        pltpu.VMEM((_BCHUNK,), jnp.float32),
        pltpu.VMEM((_BCHUNK,), jnp.float32),
        pltpu.SemaphoreType.DMA,
        pltpu.SemaphoreType.DMA,
        pltpu.SemaphoreType.DMA,
        pltpu.SemaphoreType.DMA,
    ],
)
def _mk_gather(
    xt_hbm, tt_hbm, out_hbm, plane_v, xa, xb, ra, rb, sxa, sxb, sra, srb
):
    wid = lax.axis_index("s") * 2 + lax.axis_index("c")
    xbufs, xsems = (xa, xb), (sxa, sxb)
    rbufs, rsems = (ra, rb), (sra, srb)

    def do_plane(j, carry):
        c = wid * _PLANES_PER_W + j
        f = c // _EDIM
        e = lax.rem(c, _EDIM)
        hx = pltpu.async_copy(xt_hbm.at[f, pl.ds(0, _BCHUNK)], xbufs[0], xsems[0])
        pltpu.sync_copy(tt_hbm.at[f, e], plane_v)

        row_handles = [None, None]
        for b in range(_NB):
            hx.wait()
            if b + 1 < _NB:
                hx = pltpu.async_copy(
                    xt_hbm.at[f, pl.ds((b + 1) * _BCHUNK, _BCHUNK)],
                    xbufs[(b + 1) % 2],
                    xsems[(b + 1) % 2],
                )
            if row_handles[b % 2] is not None:
                row_handles[b % 2].wait()
            xv = xbufs[b % 2]
            row_v = rbufs[b % 2]

            def gather_group(i, carry3, xv=xv, row_v=row_v):
                base = i * (_LANES * _UNROLL)
                for k in range(_UNROLL):
                    o = base + k * _LANES
                    idx = xv[pl.ds(o, _LANES)]
                    row_v[pl.ds(o, _LANES)] = plsc.load_gather(plane_v, [idx])
                return carry3

            lax.fori_loop(0, _BCHUNK // (_LANES * _UNROLL), gather_group, 0)
            row_handles[b % 2] = pltpu.async_copy(
                row_v,
                out_hbm.at[c, pl.ds(b * _BCHUNK, _BCHUNK)],
                rsems[b % 2],
            )
        row_handles[0].wait()
        row_handles[1].wait()
        return carry

    lax.fori_loop(0, _PLANES_PER_W, do_plane, 0)


def kernel(x, tables):
    xt = x.T
    tt = tables.transpose(0, 2, 1)
    out = _mk_gather(xt, tt)
    return out.T
